# baseline (speedup 1.0000x reference)
import functools

import jax
import jax.numpy as jnp
from jax import lax
from jax.experimental import pallas as pl
from jax.experimental.pallas import tpu as pltpu
from jax.experimental.pallas import tpu_sc as plsc

D_MODEL = 128
SELECT_N = 2048


def _score_t_body(b_ref, x_ref, w_ref, eye_ref, key_ref, p_ref):
    xb = x_ref[0]
    s = jax.lax.dot_general(
        w_ref[...], xb, (((1,), (0,)), ((), ())),
        preferred_element_type=jnp.float32,
    )
    s = s + b_ref[0]
    u = jax.lax.bitcast_convert_type(s, jnp.uint32)
    u = jnp.where(u == jnp.uint32(0x80000000), jnp.uint32(0), u)
    key = jnp.where(u >= jnp.uint32(0x80000000), u, (~u) ^ jnp.uint32(0x80000000))
    key_ref[0] = jax.lax.bitcast_convert_type(key, jnp.int32)
    p_ref[0] = jax.lax.dot_general(
        xb, eye_ref[...], (((0,), (0,)), ((), ())),
        precision=jax.lax.Precision.HIGHEST,
        preferred_element_type=jnp.float32,
    )


def _scores_and_patches(x, W, b):
    B, C, D, P = x.shape
    xf = x.reshape(B * C, D, P)
    eye = jnp.eye(D, dtype=jnp.float32)
    keys, patches = pl.pallas_call(
        _score_t_body,
        grid=(B * C,),
        in_specs=[
            pl.BlockSpec(memory_space=pltpu.SMEM),
            pl.BlockSpec((1, D, P), lambda i: (i, 0, 0)),
            pl.BlockSpec((1, D), lambda i: (0, 0)),
            pl.BlockSpec((D, D), lambda i: (0, 0)),
        ],
        out_specs=[
            pl.BlockSpec((1, 1, P), lambda i: (i, 0, 0)),
            pl.BlockSpec((1, P, D), lambda i: (i, 0, 0)),
        ],
        out_shape=[
            jax.ShapeDtypeStruct((B * C, 1, P), jnp.int32),
            jax.ShapeDtypeStruct((B * C, P, D), jnp.float32),
        ],
    )(b, xf, W, eye)
    return keys.reshape(B, C * P), patches.reshape(B * C * P, D)


_SC_INFO = plsc.get_sparse_core_info()
_NC, _NS = _SC_INFO.num_cores, _SC_INFO.num_subcores
_NW = _NC * _NS


_TOTAL = 32768
_NVT = _TOTAL // 16


def _topk_indices(keys):
    B = keys.shape[0]
    mesh = plsc.VectorSubcoreMesh(core_axis_name="c", subcore_axis_name="s")

    @functools.partial(
        pl.kernel, mesh=mesh,
        out_type=jax.ShapeDtypeStruct((B, SELECT_N), jnp.int32),
        compiler_params=pltpu.CompilerParams(needs_layout_passes=False),
        scratch_types=[
            pltpu.VMEM((_TOTAL,), jnp.int32),
            pltpu.VMEM((_TOTAL,), jnp.int32),
            pltpu.VMEM((_TOTAL,), jnp.int32),
            pltpu.VMEM((4096,), jnp.int32),
            pltpu.VMEM((4096,), jnp.int32),
        ],
    )
    def k(keys_hbm, out_hbm, keys_v, perm_a, perm_b, hist, offs):
        wid = lax.axis_index("s") * _NC + lax.axis_index("c")
        iota = lax.broadcasted_iota(jnp.int32, (16,), 0)
        ones = jnp.ones((16,), jnp.int32)
        zeros = jnp.zeros((16,), jnp.int32)

        @pl.when(wid < B)
        def _():
            b = wid
            pltpu.sync_copy(keys_hbm.at[b], keys_v)

            def zh(i, _):
                plsc.store_scatter(hist, [i * 16 + iota], zeros)
                return 0
            lax.fori_loop(0, 256, zh, 0)

            def h0(i, _):
                v = plsc.load_gather(keys_v, [i * 16 + iota])
                d = lax.shift_right_logical(v, 24)
                plsc.addupdate_scatter(hist, [d * 16 + iota], ones)
                return 0
            lax.fori_loop(0, _NVT, h0, 0)

            def cut(d, carry):
                cum, t = carry
                hv = plsc.load_gather(hist, [d * 16 + iota])
                new_cum = cum + jnp.sum(hv)
                t = jnp.where((t < 0) & (new_cum >= SELECT_N), d, t)
                return new_cum, t
            _, t = lax.fori_loop(
                0, 256, cut, (jnp.int32(0), jnp.int32(-1)))

            def comp(i, base):
                idxv = i * 16 + iota
                v = plsc.load_gather(keys_v, [idxv])
                d = lax.shift_right_logical(v, 24)
                p = (d <= t).astype(jnp.int32)
                c = plsc.cumsum(p)
                plsc.store_scatter(perm_a, [base + c - 1], idxv, mask=p == 1)
                return base + jnp.sum(p)
            m = lax.fori_loop(0, _NVT, comp, jnp.int32(0))

            nv = (m + 15) // 16
            mpad = nv * 16
            tz = jnp.minimum(m + iota, jnp.int32(_TOTAL - 1))
            tmask = (m + iota) < mpad
            plsc.store_scatter(perm_a, [tz], zeros, mask=tmask)
            plsc.store_scatter(perm_b, [tz], zeros, mask=tmask)

            def radix_pass(shift, src, dst):
                def z(i, _):
                    plsc.store_scatter(hist, [i * 16 + iota], zeros)
                    return 0
                lax.fori_loop(0, 256, z, 0)

                def s1(i, _):
                    j = iota * nv + i
                    pj = plsc.load_gather(src, [j])
                    kv = plsc.load_gather(keys_v, [pj])
                    d = jnp.bitwise_and(
                        lax.shift_right_logical(kv, shift), 255)
                    plsc.addupdate_scatter(
                        hist, [d * 16 + iota], ones, mask=j < m)
                    return 0
                lax.fori_loop(0, nv, s1, 0)

                def sc2(d, base):
                    hv = plsc.load_gather(hist, [d * 16 + iota])
                    ex = plsc.cumsum(hv) - hv
                    plsc.store_scatter(offs, [d * 16 + iota], base + ex)
                    return base + jnp.sum(hv)
                lax.fori_loop(0, 256, sc2, jnp.int32(0))

                def s2(i, _):
                    j = iota * nv + i
                    msk = j < m
                    pj = plsc.load_gather(src, [j])
                    kv = plsc.load_gather(keys_v, [pj])
                    d = jnp.bitwise_and(
                        lax.shift_right_logical(kv, shift), 255)
                    hidx = d * 16 + iota
                    c = plsc.load_gather(offs, [hidx])
                    plsc.store_scatter(dst, [c], pj, mask=msk)
                    plsc.addupdate_scatter(offs, [hidx], ones, mask=msk)
                    return 0
                lax.fori_loop(0, nv, s2, 0)

            radix_pass(0, perm_a, perm_b)
            radix_pass(8, perm_b, perm_a)
            radix_pass(16, perm_a, perm_b)
            radix_pass(24, perm_b, perm_a)

            def outw(i, _):
                v = plsc.load_gather(perm_a, [i * 16 + iota])
                plsc.store_scatter(
                    perm_b, [i * 16 + iota], v + b * _TOTAL)
                return 0
            lax.fori_loop(0, SELECT_N // 16, outw, 0)
            pltpu.sync_copy(perm_b.at[pl.ds(0, SELECT_N)], out_hbm.at[b])

    return k(keys)


def _gather_rows(idx_flat, patches_flat):
    n_rows = idx_flat.shape[0]
    rows_per_w = n_rows // _NW
    mesh = plsc.VectorSubcoreMesh(core_axis_name="c", subcore_axis_name="s")

    @functools.partial(
        pl.kernel, mesh=mesh,
        out_type=jax.ShapeDtypeStruct((n_rows, D_MODEL), jnp.float32),
        scratch_types=[
            pltpu.VMEM((rows_per_w,), jnp.int32),
            pltpu.VMEM((rows_per_w, D_MODEL), jnp.float32),
            pltpu.SemaphoreType.DMA,
        ],
    )
    def k(idx_hbm, patches_hbm, out_hbm, idx_v, rows_v, sem):
        wid = lax.axis_index("s") * _NC + lax.axis_index("c")
        base = wid * rows_per_w
        pltpu.sync_copy(idx_hbm.at[pl.ds(base, rows_per_w)], idx_v)
        pltpu.async_copy(patches_hbm.at[idx_v], rows_v, sem).wait()
        pltpu.sync_copy(rows_v, out_hbm.at[pl.ds(base, rows_per_w)])

    return k(idx_flat, patches_flat)


def kernel(x, W, b):
    B, C, D, P = x.shape
    keys, patches = _scores_and_patches(x, W, b)
    gidx = _topk_indices(keys)
    selected = _gather_rows(gidx.reshape(-1), patches)
    return selected.reshape(B, SELECT_N, D)
